# baseline (speedup 1.0000x reference)
import jax
import jax.numpy as jnp
from jax import lax
from jax.experimental import pallas as pl
from jax.experimental.pallas import tpu as pltpu
from jax.experimental.pallas import tpu_sc as plsc

SIGMA = 5.0
B, C, H, W = 128, 3, 384, 384
NC, NS, L = 2, 16, 16
NW = NC * NS
SR = 8
NSTRIPE = H // SR
INV = 1.0 / (2.0 * SIGMA * SIGMA)
KW = W // L


def _sc_body(mask_ref, gtx_ref, gty_ref, out_ref, mv_ref, xv_ref, yv_ref, buf_ref):
    wid = lax.axis_index("s") * NC + lax.axis_index("c")
    pltpu.sync_copy(mask_ref, mv_ref)
    pltpu.sync_copy(gtx_ref, xv_ref)
    pltpu.sync_copy(gty_ref, yv_ref)

    def step(b, carry):
        m = mv_ref[pl.ds(b, L)][0]

        @pl.when(m != 0.0)
        def _():
            x0 = xv_ref[pl.ds(b, L)][0]
            y0 = yv_ref[pl.ds(b, L)][0]
            gx = []
            for k in range(KW):
                dx = (lax.iota(jnp.int32, L) + (k * L)).astype(jnp.float32) - x0
                gx.append(jnp.exp(-(dx * dx) * INV))
            for si in range(NSTRIPE // NW + 1):
                stripe = wid + si * NW

                @pl.when(stripe < NSTRIPE)
                def _():
                    r0 = stripe * SR
                    dy = (lax.iota(jnp.int32, L) + r0).astype(jnp.float32) - y0
                    gy = jnp.exp(-(dy * dy) * INV)
                    for j in range(SR):
                        gyj = gy[j]
                        for k in range(KW):
                            buf_ref[j, pl.ds(k * L, L)] = gyj * gx[k]
                    pltpu.sync_copy(buf_ref, out_ref.at[b, 0, pl.ds(r0, SR)])

        return carry

    lax.fori_loop(0, B, step, jnp.int32(0))


def _make_sc_scatter():
    return pl.kernel(
        _sc_body,
        out_type=(),
        mesh=plsc.VectorSubcoreMesh(core_axis_name="c", subcore_axis_name="s"),
        scratch_types=[
            pltpu.VMEM((B + L,), jnp.float32),
            pltpu.VMEM((B + L,), jnp.float32),
            pltpu.VMEM((B + L,), jnp.float32),
            pltpu.VMEM((SR, W), jnp.float32),
        ],
    )


def kernel(images, gt, mask):
    pad = jnp.zeros((L,), jnp.float32)
    mask_f = jnp.concatenate([mask.astype(jnp.float32), pad])
    gtx = jnp.concatenate([gt[:, 0], pad])
    gty = jnp.concatenate([gt[:, 1], pad])
    out_ref = jax.new_ref(images)
    _make_sc_scatter()(mask_f, gtx, gty, out_ref)
    return out_ref[...]


# speedup vs baseline: 1.1218x; 1.1218x over previous
import jax
import jax.numpy as jnp
from jax import lax
from jax.experimental import pallas as pl
from jax.experimental.pallas import tpu as pltpu
from jax.experimental.pallas import tpu_sc as plsc

SIGMA = 5.0
B, C, H, W = 128, 3, 384, 384
NC, NS, L = 2, 16, 16
NW = NC * NS
SR = 8
NSTRIPE = H // SR
INV = 1.0 / (2.0 * SIGMA * SIGMA)
KW = W // L


def _sc_body(meta_ref, xv_hbm, yv_hbm, out_ref,
             mv_ref, xv_ref, yv_ref, buf_ref, sem0, sem1):
    wid = lax.axis_index("s") * NC + lax.axis_index("c")
    pltpu.sync_copy(meta_ref, mv_ref)
    pltpu.sync_copy(xv_hbm, xv_ref)
    pltpu.sync_copy(yv_hbm, yv_ref)
    n = mv_ref[pl.ds(0, L)][0]
    sems = (sem0, sem1)

    def stripe_dst(b, si):
        r0 = (wid + si * NW) * SR
        return out_ref.at[b, 0, pl.ds(r0, SR)]

    def step(j, carry):
        b = mv_ref[pl.ds(1 + j, L)][0]
        x0 = xv_ref[pl.ds(b, L)][0]
        y0 = yv_ref[pl.ds(b, L)][0]
        slot = lax.rem(j, 2)

        gx = []
        for k in range(KW):
            dx = (lax.iota(jnp.int32, L) + (k * L)).astype(jnp.float32) - x0
            gx.append(jnp.exp(-(dx * dx) * INV))

        for si in range(2):
            stripe_ok = (wid + si * NW) < NSTRIPE

            @pl.when(stripe_ok)
            def _():
                @pl.when(j >= 2)
                def _():
                    for s in range(2):
                        @pl.when(slot == s)
                        def _():
                            pltpu.make_async_copy(
                                buf_ref.at[s, si], stripe_dst(0, si),
                                sems[s]).wait()

                r0 = (wid + si * NW) * SR
                dy = (lax.iota(jnp.int32, L) + r0).astype(jnp.float32) - y0
                gy = jnp.exp(-(dy * dy) * INV)
                for s in range(2):
                    @pl.when(slot == s)
                    def _():
                        for jj in range(SR):
                            gyj = gy[jj]
                            for k in range(KW):
                                buf_ref[s, si, jj, pl.ds(k * L, L)] = gyj * gx[k]
                        pltpu.make_async_copy(
                            buf_ref.at[s, si], stripe_dst(b, si),
                            sems[s]).start()
        return carry

    lax.fori_loop(0, n, step, jnp.int32(0))

    for d in range(2):
        @pl.when(n > d)
        def _():
            slot = lax.rem(n - 1 - d, 2)
            for si in range(2):
                @pl.when((wid + si * NW) < NSTRIPE)
                def _():
                    for s in range(2):
                        @pl.when(slot == s)
                        def _():
                            pltpu.make_async_copy(
                                buf_ref.at[s, si], stripe_dst(0, si),
                                sems[s]).wait()


def _make_sc_scatter():
    return pl.kernel(
        _sc_body,
        out_type=(),
        mesh=plsc.VectorSubcoreMesh(core_axis_name="c", subcore_axis_name="s"),
        scratch_types=[
            pltpu.VMEM((1 + B + L,), jnp.int32),
            pltpu.VMEM((B + L,), jnp.float32),
            pltpu.VMEM((B + L,), jnp.float32),
            pltpu.VMEM((2, 2, SR, W), jnp.float32),
            pltpu.SemaphoreType.DMA,
            pltpu.SemaphoreType.DMA,
        ],
    )


def kernel(images, gt, mask):
    idx = jnp.nonzero(mask, size=B, fill_value=0)[0].astype(jnp.int32)
    n = jnp.sum(mask.astype(jnp.int32))
    meta = jnp.concatenate([n[None], idx, jnp.zeros((L,), jnp.int32)])
    pad = jnp.zeros((L,), jnp.float32)
    gtx = jnp.concatenate([gt[:, 0], pad])
    gty = jnp.concatenate([gt[:, 1], pad])
    out_ref = jax.new_ref(images)
    _make_sc_scatter()(meta, gtx, gty, out_ref)
    return out_ref[...]


# speedup vs baseline: 1.3975x; 1.2457x over previous
import jax
import jax.numpy as jnp
from jax import lax
from jax.experimental import pallas as pl
from jax.experimental.pallas import tpu as pltpu

SIGMA = 5.0
B, C, H, W = 128, 3, 384, 384
INV = 1.0 / (2.0 * SIGMA * SIGMA)
BB = 8


def _body(mask_ref, gt_ref, img_ref, out_ref):
    i = pl.program_id(0)
    out_ref[...] = img_ref[...]

    for j in range(BB):
        b = i * BB + j

        @pl.when(mask_ref[b] != 0)
        def _():
            x0 = gt_ref[b, 0]
            y0 = gt_ref[b, 1]
            xs = lax.broadcasted_iota(jnp.int32, (1, W), 1).astype(jnp.float32)
            ys = lax.broadcasted_iota(jnp.int32, (H, 1), 0).astype(jnp.float32)
            gx = jnp.exp(-((xs - x0) ** 2) * INV)
            gy = jnp.exp(-((ys - y0) ** 2) * INV)
            out_ref[j, 0] = gy * gx


def kernel(images, gt, mask):
    mask_i = mask.astype(jnp.int32)
    return pl.pallas_call(
        _body,
        grid=(B // BB,),
        in_specs=[
            pl.BlockSpec(memory_space=pltpu.SMEM),
            pl.BlockSpec(memory_space=pltpu.SMEM),
            pl.BlockSpec((BB, C, H, W), lambda i: (i, 0, 0, 0)),
        ],
        out_specs=pl.BlockSpec((BB, C, H, W), lambda i: (i, 0, 0, 0)),
        out_shape=jax.ShapeDtypeStruct((B, C, H, W), jnp.float32),
    )(mask_i, gt, images)
